# Initial kernel scaffold; baseline (speedup 1.0000x reference)
import jax
import jax.numpy as jnp
from jax.experimental import pallas as pl


def kernel(x, edge_index, batch, W_feat, W_convs, b_convs, W_ea, b_ea, W_na, b_na, W_ctx, b_ctx, W_obj, b_obj, fc1_c_W, fc1_c_b, fc2_c_W, fc2_c_b, fc1_o_W, fc1_o_b, fc2_o_W, fc2_o_b, fc1_co_W, fc1_co_b, fc2_co_W, fc2_co_b):
    raise NotImplementedError("write your pallas kernel here")


# speedup vs baseline: 12.6442x; 12.6442x over previous
import functools

import jax
import jax.numpy as jnp
from jax import lax
from jax.experimental import pallas as pl
from jax.experimental.pallas import tpu as pltpu
from jax.experimental.pallas import tpu_sc as plsc

N = 10000
E = 320000
D = 128
H = 128
C = 10
G = 128
EPS = 1e-5

NP2 = 10240
NPOOL = 10112
PADSTART = 10112
EP = 335872
NC = 2
NS = 16
NW = NC * NS
CH = 128
PE_TILE = EP // NW
NCHT = PE_TILE // CH
PE_CTILE = EP // NS
NCHC = PE_CTILE // CH
ROWS32 = NP2 // NW
ROWS16 = NP2 // NS
POOL_PT = NPOOL // NS
PG = G + 8

_mesh = lambda: plsc.VectorSubcoreMesh(core_axis_name="c", subcore_axis_name="s",
                                       num_cores=NC, num_subcores=NS)

_i16 = lambda: jnp.arange(16, dtype=jnp.int32)
_z16i = lambda: jnp.zeros((16,), jnp.int32)
_z16f = lambda: jnp.zeros((16,), jnp.float32)


def _rsqrt16(x):
    i = plsc.bitcast(x, jnp.int32)
    y = plsc.bitcast(jnp.int32(0x5F3759DF) - (i >> 1), jnp.float32)
    for _ in range(3):
        y = y * (1.5 - 0.5 * x * y * y)
    return y


def _zero_rows(ref, nrows, width):
    z = _z16f()

    def body(r, carry):
        for j in range(width // 16):
            ref[r, pl.ds(j * 16, 16)] = z
        return carry

    lax.fori_loop(0, nrows, body, 0)


def _fill1d(ref, n, value):
    v = jnp.full((16,), value, jnp.float32)

    def body(i, carry):
        ref[pl.ds(i * 16, 16)] = v
        return carry

    lax.fori_loop(0, n // 16, body, 0)


def _make_sc_prep(interpret=False):
    @functools.partial(
        pl.kernel,
        out_type=jax.ShapeDtypeStruct((NC, NP2), jnp.float32),
        mesh=_mesh(),
        compiler_params=pltpu.CompilerParams(needs_layout_passes=False),
        scratch_types=[
            pltpu.VMEM((CH,), jnp.int32),
            pltpu.VMEM((CH,), jnp.float32),
            pltpu.VMEM((ROWS16,), jnp.float32),
            pltpu.VMEM_SHARED((NP2,), jnp.float32),
        ],
        interpret=interpret,
    )
    def k(row_hbm, out_hbm, idx_v, ones_v, zbuf, deg_sh):
        cid = lax.axis_index("c")
        sid = lax.axis_index("s")
        wid = sid * NC + cid
        _fill1d(ones_v, CH, 1.0)
        _fill1d(zbuf, ROWS16, 0.0)
        r0 = sid * ROWS16
        pltpu.sync_copy(zbuf, deg_sh.at[pl.ds(r0, ROWS16)])
        plsc.subcore_barrier()

        base = wid * PE_TILE

        def body(ch, carry):
            pltpu.sync_copy(row_hbm.at[pl.ds(base + ch * CH, CH)], idx_v)
            pltpu.sync_copy(ones_v, deg_sh.at[idx_v], add=True)
            return carry

        lax.fori_loop(0, NCHT, body, 0)
        plsc.subcore_barrier()
        pltpu.sync_copy(deg_sh.at[pl.ds(r0, ROWS16)],
                        out_hbm.at[cid, pl.ds(r0, ROWS16)])

    return k


def _make_sc_conv(interpret=False):
    @functools.partial(
        pl.kernel,
        out_type=jax.ShapeDtypeStruct((NC, NP2, H), jnp.float32),
        mesh=_mesh(),
        compiler_params=pltpu.CompilerParams(needs_layout_passes=False),
        scratch_types=[
            pltpu.VMEM((CH,), jnp.int32),
            pltpu.VMEM((CH,), jnp.int32),
            pltpu.VMEM((CH, H), jnp.float32),
            pltpu.VMEM((CH, H), jnp.float32),
            pltpu.VMEM_SHARED((NP2, H), jnp.float32),
            pltpu.SemaphoreType.DMA,
            pltpu.SemaphoreType.DMA,
        ],
        interpret=interpret,
    )
    def k(tab_hbm, row_hbm, col_hbm, out_hbm, idxr, idxc, bufA, bufB, acc_sh,
          semA, semB):
        cid = lax.axis_index("c")
        sid = lax.axis_index("s")
        wid = sid * NC + cid
        _zero_rows(bufA, CH, H)
        r0 = sid * ROWS16
        for b in range(ROWS16 // CH):
            pltpu.sync_copy(bufA, acc_sh.at[pl.ds(r0 + b * CH, CH)])
        plsc.subcore_barrier()

        base = wid * PE_TILE

        def body(ch, carry):
            e0 = base + ch * CH
            pltpu.sync_copy(row_hbm.at[pl.ds(e0, CH)], idxr)
            pltpu.sync_copy(col_hbm.at[pl.ds(e0, CH)], idxc)
            pltpu.async_copy(tab_hbm.at[idxr], bufA, semA).wait()
            pltpu.sync_copy(bufA, acc_sh.at[idxc], add=True)
            return carry

        lax.fori_loop(0, NCHT, body, 0)
        plsc.subcore_barrier()
        pltpu.sync_copy(acc_sh.at[pl.ds(r0, ROWS16)],
                        out_hbm.at[cid, pl.ds(r0, ROWS16)])

    return k


def _make_sc_att(interpret=False):
    @functools.partial(
        pl.kernel,
        out_type=(
            jax.ShapeDtypeStruct((NC, EP), jnp.float32),
            jax.ShapeDtypeStruct((NC, NP2), jnp.float32),
        ),
        mesh=_mesh(),
        compiler_params=pltpu.CompilerParams(needs_layout_passes=False),
        scratch_types=[
            pltpu.VMEM((NP2,), jnp.float32),
            pltpu.VMEM((NP2,), jnp.float32),
            pltpu.VMEM((NP2,), jnp.float32),
            pltpu.VMEM((NP2,), jnp.float32),
            pltpu.VMEM((CH,), jnp.int32),
            pltpu.VMEM((CH,), jnp.int32),
            pltpu.VMEM((PE_CTILE,), jnp.float32),
            pltpu.VMEM((ROWS16,), jnp.float32),
            pltpu.VMEM((ROWS16,), jnp.float32),
            pltpu.VMEM((NP2,), jnp.float32),
            pltpu.VMEM((CH,), jnp.float32),
            pltpu.VMEM_SHARED((NP2,), jnp.float32),
            pltpu.VMEM_SHARED((NP2,), jnp.float32),
        ],
        interpret=interpret,
    )
    def k(att0_hbm, att1_hbm, att2_hbm, att3_hbm, row_hbm, col_hbm,
          alpha_hbm, dinv_hbm,
          a0_v, a1_v, b0_v, b1_v, idxr, idxc, ew_all, degbuf, dinv1d,
          dinv_v, abuf, deg_sh, dinv_sh):
        cid = lax.axis_index("c")
        sid = lax.axis_index("s")
        r0 = sid * ROWS16
        _fill1d(degbuf, ROWS16, 0.0)
        pltpu.sync_copy(degbuf, deg_sh.at[pl.ds(r0, ROWS16)])
        pltpu.sync_copy(att0_hbm, a0_v)
        pltpu.sync_copy(att1_hbm, a1_v)
        pltpu.sync_copy(att2_hbm, b0_v)
        pltpu.sync_copy(att3_hbm, b1_v)
        plsc.subcore_barrier()

        base = sid * PE_CTILE
        i16 = _i16()
        is_c = (cid == 0)

        def pass1(ch, carry):
            e0 = base + ch * CH
            pltpu.sync_copy(row_hbm.at[pl.ds(e0, CH)], idxr)
            pltpu.sync_copy(col_hbm.at[pl.ds(e0, CH)], idxc)
            for g in range(CH // 16):
                r16 = idxr[pl.ds(g * 16, 16)]
                c16 = idxc[pl.ds(g * 16, 16)]
                a0 = plsc.load_gather(a0_v, [r16])
                a1 = plsc.load_gather(a1_v, [r16])
                b0 = plsc.load_gather(b0_v, [c16])
                b1 = plsc.load_gather(b1_v, [c16])
                l0 = a0 + b0
                l1 = a1 + b1
                m = jnp.maximum(l0, l1)
                x0 = jnp.exp(l0 - m)
                x1 = jnp.exp(l1 - m)
                s = x0 + x1
                ew = jnp.where(is_c, x0, x1) / s
                eid = e0 + g * 16 + i16
                ew = jnp.where(eid < E, ew,
                               jnp.where(eid < E + N, 1.0, 0.0))
                ew_all[pl.ds(ch * CH + g * 16, 16)] = ew
            pltpu.sync_copy(ew_all.at[pl.ds(ch * CH, CH)],
                            deg_sh.at[idxr], add=True)
            return carry

        lax.fori_loop(0, NCHC, pass1, 0)
        plsc.subcore_barrier()

        pltpu.sync_copy(deg_sh.at[pl.ds(r0, ROWS16)], degbuf)
        for g in range(ROWS16 // 16):
            dg = degbuf[pl.ds(g * 16, 16)]
            y = jnp.where(dg > 0, _rsqrt16(dg), 0.0)
            dinv1d[pl.ds(g * 16, 16)] = y
        pltpu.sync_copy(dinv1d, dinv_sh.at[pl.ds(r0, ROWS16)])
        pltpu.sync_copy(dinv1d, dinv_hbm.at[cid, pl.ds(r0, ROWS16)])
        plsc.subcore_barrier()
        pltpu.sync_copy(dinv_sh, dinv_v)

        def pass2(ch, carry):
            e0 = base + ch * CH
            pltpu.sync_copy(row_hbm.at[pl.ds(e0, CH)], idxr)
            for g in range(CH // 16):
                r16 = idxr[pl.ds(g * 16, 16)]
                dv = plsc.load_gather(dinv_v, [r16])
                ew = ew_all[pl.ds(ch * CH + g * 16, 16)]
                abuf[pl.ds(g * 16, 16)] = ew * dv
            pltpu.sync_copy(abuf, alpha_hbm.at[cid, pl.ds(e0, CH)])
            return carry

        lax.fori_loop(0, NCHC, pass2, 0)

    return k


def _make_sc_conv_co(interpret=False):
    @functools.partial(
        pl.kernel,
        out_type=jax.ShapeDtypeStruct((NC, NP2, H), jnp.float32),
        mesh=_mesh(),
        compiler_params=pltpu.CompilerParams(needs_layout_passes=False),
        scratch_types=[
            pltpu.VMEM((CH,), jnp.int32),
            pltpu.VMEM((CH,), jnp.int32),
            pltpu.VMEM((CH,), jnp.float32),
            pltpu.VMEM((CH, H), jnp.float32),
            pltpu.VMEM_SHARED((NP2, H), jnp.float32),
            pltpu.SemaphoreType.DMA,
        ],
        interpret=interpret,
    )
    def k(xw_hbm, alpha_hbm, row_hbm, col_hbm,
          out_hbm, idxr, idxc, avmem, bufA, acc_sh, semA):
        cid = lax.axis_index("c")
        sid = lax.axis_index("s")
        _zero_rows(bufA, CH, H)
        r0 = sid * ROWS16
        for b in range(ROWS16 // CH):
            pltpu.sync_copy(bufA, acc_sh.at[pl.ds(r0 + b * CH, CH)])
        plsc.subcore_barrier()

        base = sid * PE_CTILE

        def body(ch, carry):
            e0 = base + ch * CH
            pltpu.sync_copy(row_hbm.at[pl.ds(e0, CH)], idxr)
            pltpu.sync_copy(col_hbm.at[pl.ds(e0, CH)], idxc)
            pltpu.sync_copy(alpha_hbm.at[cid, pl.ds(e0, CH)], avmem)
            pltpu.async_copy(xw_hbm.at[cid].at[idxr], bufA, semA).wait()

            def scale(r, carry2):
                al = plsc.load_gather(avmem, [jnp.full((16,), r, jnp.int32)])
                for j in range(H // 16):
                    bufA[r, pl.ds(j * 16, 16)] = bufA[r, pl.ds(j * 16, 16)] * al
                return carry2

            lax.fori_loop(0, CH, scale, 0)
            pltpu.sync_copy(bufA, acc_sh.at[idxc], add=True)
            return carry

        lax.fori_loop(0, NCHC, body, 0)
        plsc.subcore_barrier()
        pltpu.sync_copy(acc_sh.at[pl.ds(r0, ROWS16)],
                        out_hbm.at[cid, pl.ds(r0, ROWS16)])

    return k


def _bn(x):
    m = jnp.mean(x, axis=0)
    xc = x - m
    v = jnp.mean(xc * xc, axis=0)
    return xc * lax.rsqrt(v + EPS) + 1e-4


def _dinv_from(deg_ref):
    d = deg_ref[...]
    ds_ = (d[0] + d[1]).reshape(NP2, 1)
    return jnp.where(ds_ > 0, lax.rsqrt(ds_), 0.0)[:N]


def _padz(x):
    return jnp.concatenate([x, jnp.zeros((NP2 - N, x.shape[1]), x.dtype)], 0)


def _make_tc_layer1(interpret=False):
    def body(x_ref, wf_ref, w0_ref, deg_ref, o_ref):
        xb = _bn(x_ref[...])
        h0 = jnp.maximum(jnp.dot(xb, wf_ref[...],
                                 preferred_element_type=jnp.float32), 0.0)
        xw = jnp.dot(_bn(h0), w0_ref[...], preferred_element_type=jnp.float32)
        o_ref[...] = _padz(_dinv_from(deg_ref) * xw)

    return pl.pallas_call(
        body, out_shape=jax.ShapeDtypeStruct((NP2, H), jnp.float32),
        interpret=interpret)


def _make_tc_layer(interpret=False):
    def body(t_ref, deg_ref, b_ref, w_ref, o_ref):
        dinv = _dinv_from(deg_ref)
        s = t_ref[0, :N, :] + t_ref[1, :N, :]
        h = jnp.maximum(dinv * s + b_ref[...], 0.0)
        xw = jnp.dot(_bn(h), w_ref[...], preferred_element_type=jnp.float32)
        o_ref[...] = _padz(dinv * xw)

    return pl.pallas_call(
        body, out_shape=jax.ShapeDtypeStruct((NP2, H), jnp.float32),
        interpret=interpret)


def _make_tc_att(interpret=False):
    def body(t_ref, deg_ref, b_ref, weaT_ref, bea_ref, wna_ref, bna_ref,
             wctx_ref, wobj_ref, a0_ref, a1_ref, b0_ref, b1_ref, xw_ref):
        dinv = _dinv_from(deg_ref)
        s = t_ref[0, :N, :] + t_ref[1, :N, :]
        h3 = jnp.maximum(dinv * s + b_ref[...], 0.0)
        weaT = weaT_ref[...]
        bea = bea_ref[...]
        zrow = jnp.zeros((NP2 - N,), jnp.float32)

        def rowdot(w_1x128):
            r = lax.dot_general(w_1x128, h3, (((1,), (1,)), ((), ())),
                                preferred_element_type=jnp.float32)
            return r.reshape(N)

        a0_ref[...] = jnp.concatenate([rowdot(weaT[0:1, :H]) + bea[0], zrow])
        a1_ref[...] = jnp.concatenate([rowdot(weaT[1:2, :H]) + bea[1], zrow])
        b0_ref[...] = jnp.concatenate([rowdot(weaT[0:1, H:]), zrow])
        b1_ref[...] = jnp.concatenate([rowdot(weaT[1:2, H:]), zrow])
        nl = jnp.dot(h3, wna_ref[...], preferred_element_type=jnp.float32) \
            + bna_ref[...]
        m = jnp.max(nl, axis=1, keepdims=True)
        ex = jnp.exp(nl - m)
        nat = ex / jnp.sum(ex, axis=1, keepdims=True)
        xw_ref[0] = _padz(jnp.dot(_bn(nat[:, 0:1] * h3), wctx_ref[...],
                                  preferred_element_type=jnp.float32))
        xw_ref[1] = _padz(jnp.dot(_bn(nat[:, 1:2] * h3), wobj_ref[...],
                                  preferred_element_type=jnp.float32))

    return pl.pallas_call(
        body,
        out_shape=(jax.ShapeDtypeStruct((NP2,), jnp.float32),
                   jax.ShapeDtypeStruct((NP2,), jnp.float32),
                   jax.ShapeDtypeStruct((NP2,), jnp.float32),
                   jax.ShapeDtypeStruct((NP2,), jnp.float32),
                   jax.ShapeDtypeStruct((NC, NP2, H), jnp.float32)),
        interpret=interpret)


def _make_tc_final(interpret=False):
    def body(t_ref, dinv_ref, bias_ref, batch_ref, w1c, b1c, w2c, b2c,
             w1o, b1o, w2o, b2o, w1co, b1co, w2co, b2co,
             oc_ref, oo_ref, oco_ref):
        bt = batch_ref[...]
        onehotT = (lax.broadcasted_iota(jnp.int32, (G, N), 0)
                   == jnp.broadcast_to(bt, (G, N))).astype(jnp.float32)
        dinv = dinv_ref[...]
        bias = bias_ref[...]

        def chan_pool(c2):
            t = t_ref[c2, :N, :]
            dv = dinv[c2].reshape(NP2, 1)[:N]
            xr = jnp.maximum(dv * t + bias[c2], 0.0)
            return jnp.dot(onehotT, xr, preferred_element_type=jnp.float32)

        pc = chan_pool(0)
        po = chan_pool(1)

        def readout(z, w1, b1, w2, b2):
            z = _bn(z)
            z = jnp.maximum(jnp.dot(z, w1[...],
                                    preferred_element_type=jnp.float32)
                            + b1[...], 0.0)
            z = _bn(z)
            z = jnp.dot(z, w2[...], preferred_element_type=jnp.float32) \
                + b2[...]
            m = jnp.max(z, axis=1, keepdims=True)
            return z - (m + jnp.log(jnp.sum(jnp.exp(z - m), axis=1,
                                            keepdims=True)))

        oc_ref[...] = readout(pc, w1c, b1c, w2c, b2c)
        oo_ref[...] = readout(po, w1o, b1o, w2o, b2o)
        oco_ref[...] = readout(pc + po, w1co, b1co, w2co, b2co)

    return pl.pallas_call(
        body,
        out_shape=(jax.ShapeDtypeStruct((G, C), jnp.float32),
                   jax.ShapeDtypeStruct((G, C), jnp.float32),
                   jax.ShapeDtypeStruct((G, C), jnp.float32)),
        interpret=interpret)


_sc_cache = {}


def _sc(name, maker):
    if name not in _sc_cache:
        _sc_cache[name] = maker()
    return _sc_cache[name]


_tc_layer1 = _make_tc_layer1()
_tc_layer = _make_tc_layer()
_tc_att = _make_tc_att()
_tc_final = _make_tc_final()


def _kernel_staged(x, edge_index, batch, W_feat, W_convs, b_convs, W_ea, b_ea,
                   W_na, b_na, W_ctx, b_ctx, W_obj, b_obj, fc1_c_W, fc1_c_b,
                   fc2_c_W, fc2_c_b, fc1_o_W, fc1_o_b, fc2_o_W, fc2_o_b,
                   fc1_co_W, fc1_co_b, fc2_co_W, fc2_co_b, depth=99):
    loops = jnp.arange(N, dtype=jnp.int32)
    pade = EP - (E + N)
    padi = PADSTART + (jnp.arange(pade, dtype=jnp.int32) % 64)
    row_full = jnp.concatenate([edge_index[0], loops, padi])
    col_full = jnp.concatenate([edge_index[1], loops, padi])
    bias_st = jnp.stack([b_ctx, b_obj])
    sc_prep = _sc("prep", _make_sc_prep)
    sc_conv = _sc("conv", _make_sc_conv)
    sc_att = _sc("att", _make_sc_att)
    sc_conv_co = _sc("conv_co", _make_sc_conv_co)
    deg = sc_prep(row_full)
    if depth <= 1:
        return deg
    t = _tc_layer1(x, W_feat, W_convs[0], deg)
    T = sc_conv(t, row_full, col_full)
    if depth <= 2:
        return T
    t = _tc_layer(T, deg, b_convs[0], W_convs[1])
    T = sc_conv(t, row_full, col_full)
    t = _tc_layer(T, deg, b_convs[1], W_convs[2])
    T = sc_conv(t, row_full, col_full)
    if depth <= 3:
        return T
    a0, a1, b0, b1, xw_st = _tc_att(T, deg, b_convs[2], W_ea.T, b_ea,
                                    W_na, b_na, W_ctx, W_obj)
    alpha, dinv_co = sc_att(a0, a1, b0, b1, row_full, col_full)
    if depth <= 4:
        return alpha, dinv_co
    Tco = sc_conv_co(xw_st, alpha, row_full, col_full)
    if depth <= 5:
        return Tco
    return _tc_final(Tco, dinv_co, bias_st, batch,
                     fc1_c_W, fc1_c_b, fc2_c_W, fc2_c_b,
                     fc1_o_W, fc1_o_b, fc2_o_W, fc2_o_b,
                     fc1_co_W, fc1_co_b, fc2_co_W, fc2_co_b)


def kernel(x, edge_index, batch, W_feat, W_convs, b_convs, W_ea, b_ea, W_na,
           b_na, W_ctx, b_ctx, W_obj, b_obj, fc1_c_W, fc1_c_b, fc2_c_W,
           fc2_c_b, fc1_o_W, fc1_o_b, fc2_o_W, fc2_o_b, fc1_co_W, fc1_co_b,
           fc2_co_W, fc2_co_b):
    loops = jnp.arange(N, dtype=jnp.int32)
    pade = EP - (E + N)
    padi = PADSTART + (jnp.arange(pade, dtype=jnp.int32) % 64)
    row_full = jnp.concatenate([edge_index[0], loops, padi])
    col_full = jnp.concatenate([edge_index[1], loops, padi])
    bias_st = jnp.stack([b_ctx, b_obj])

    sc_prep = _sc("prep", _make_sc_prep)
    sc_conv = _sc("conv", _make_sc_conv)
    sc_att = _sc("att", _make_sc_att)
    sc_conv_co = _sc("conv_co", _make_sc_conv_co)

    deg = sc_prep(row_full)
    t = _tc_layer1(x, W_feat, W_convs[0], deg)
    T = sc_conv(t, row_full, col_full)
    t = _tc_layer(T, deg, b_convs[0], W_convs[1])
    T = sc_conv(t, row_full, col_full)
    t = _tc_layer(T, deg, b_convs[1], W_convs[2])
    T = sc_conv(t, row_full, col_full)
    a0, a1, b0, b1, xw_st = _tc_att(T, deg, b_convs[2], W_ea.T, b_ea,
                                    W_na, b_na, W_ctx, W_obj)
    alpha, dinv_co = sc_att(a0, a1, b0, b1, row_full, col_full)
    Tco = sc_conv_co(xw_st, alpha, row_full, col_full)
    return _tc_final(Tco, dinv_co, bias_st, batch,
                     fc1_c_W, fc1_c_b, fc2_c_W, fc2_c_b,
                     fc1_o_W, fc1_o_b, fc2_o_W, fc2_o_b,
                     fc1_co_W, fc1_co_b, fc2_co_W, fc2_co_b)


# speedup vs baseline: 18.1303x; 1.4339x over previous
import functools

import jax
import jax.numpy as jnp
from jax import lax
from jax.experimental import pallas as pl
from jax.experimental.pallas import tpu as pltpu
from jax.experimental.pallas import tpu_sc as plsc

N = 10000
E = 320000
D = 128
H = 128
C = 10
G = 128
EPS = 1e-5

NP2 = 10240
NPOOL = 10112
PADSTART = 10112
EP = 360448
NC = 2
NS = 16
NW = NC * NS
CH = 128
PE_TILE = EP // NW
NCHT = PE_TILE // CH
PE_CTILE = EP // NS
NCHC = PE_CTILE // CH
ROWS32 = NP2 // NW
ROWS16 = NP2 // NS
POOL_PT = NPOOL // NS
PG = G + 8

_mesh = lambda: plsc.VectorSubcoreMesh(core_axis_name="c", subcore_axis_name="s",
                                       num_cores=NC, num_subcores=NS)

_i16 = lambda: jnp.arange(16, dtype=jnp.int32)
_z16i = lambda: jnp.zeros((16,), jnp.int32)
_z16f = lambda: jnp.zeros((16,), jnp.float32)


def _rsqrt16(x):
    i = plsc.bitcast(x, jnp.int32)
    y = plsc.bitcast(jnp.int32(0x5F3759DF) - (i >> 1), jnp.float32)
    for _ in range(3):
        y = y * (1.5 - 0.5 * x * y * y)
    return y


def _zero_rows(ref, nrows, width):
    z = _z16f()

    def body(r, carry):
        for j in range(width // 16):
            ref[r, pl.ds(j * 16, 16)] = z
        return carry

    lax.fori_loop(0, nrows, body, 0)


def _fill1d(ref, n, value):
    v = jnp.full((16,), value, jnp.float32)

    def body(i, carry):
        ref[pl.ds(i * 16, 16)] = v
        return carry

    lax.fori_loop(0, n // 16, body, 0)


def _make_sc_prep(interpret=False):
    @functools.partial(
        pl.kernel,
        out_type=jax.ShapeDtypeStruct((NC, NP2), jnp.float32),
        mesh=_mesh(),
        compiler_params=pltpu.CompilerParams(needs_layout_passes=False),
        scratch_types=[
            pltpu.VMEM((NCHT, CH), jnp.int32),
            pltpu.VMEM((CH,), jnp.float32),
            pltpu.VMEM((ROWS16,), jnp.float32),
            pltpu.VMEM_SHARED((NP2,), jnp.float32),
            pltpu.SemaphoreType.DMA,
        ],
        interpret=interpret,
    )
    def k(row_hbm, out_hbm, rowidx, ones_v, zbuf, deg_sh, semD):
        cid = lax.axis_index("c")
        sid = lax.axis_index("s")
        wid = sid * NC + cid
        _fill1d(ones_v, CH, 1.0)
        _fill1d(zbuf, ROWS16, 0.0)
        r0 = sid * ROWS16
        pltpu.sync_copy(zbuf, deg_sh.at[pl.ds(r0, ROWS16)])
        pltpu.sync_copy(row_hbm.at[pl.ds(wid * NCHT, NCHT)], rowidx)
        plsc.subcore_barrier()

        def body(q, carry):
            for b in range(2):
                pltpu.async_copy(ones_v, deg_sh.at[rowidx.at[q * 2 + b]],
                                 semD, add=True)
            for b in range(2):
                pltpu.make_async_copy(ones_v, deg_sh.at[rowidx.at[q * 2 + b]],
                                      semD).wait()
            return carry

        lax.fori_loop(0, NCHT // 2, body, 0)
        plsc.subcore_barrier()
        pltpu.sync_copy(deg_sh.at[pl.ds(r0, ROWS16)],
                        out_hbm.at[cid, pl.ds(r0, ROWS16)])

    return k


def _make_sc_conv(interpret=False):
    @functools.partial(
        pl.kernel,
        out_type=jax.ShapeDtypeStruct((NC, NP2, H), jnp.float32),
        mesh=_mesh(),
        compiler_params=pltpu.CompilerParams(needs_layout_passes=False),
        scratch_types=[
            pltpu.VMEM((NCHT, CH), jnp.int32),
            pltpu.VMEM((CH,), jnp.int32),
            pltpu.VMEM((CH,), jnp.int32),
            pltpu.VMEM((CH, H), jnp.float32),
            pltpu.VMEM((CH, H), jnp.float32),
            pltpu.VMEM_SHARED((NP2, H), jnp.float32),
            pltpu.SemaphoreType.DMA,
            pltpu.SemaphoreType.DMA,
        ],
        interpret=interpret,
    )
    def k(tab_hbm, row_hbm, col_hbm, out_hbm, rowidx, idxcA, idxcB,
          bufA, bufB, acc_sh, semA, semB):
        cid = lax.axis_index("c")
        sid = lax.axis_index("s")
        wid = sid * NC + cid
        _zero_rows(bufA, CH, H)
        r0 = sid * ROWS16
        for b in range(ROWS16 // CH):
            pltpu.sync_copy(bufA, acc_sh.at[pl.ds(r0 + b * CH, CH)])
        cb = wid * NCHT
        pltpu.sync_copy(row_hbm.at[pl.ds(cb, NCHT)], rowidx)
        plsc.subcore_barrier()

        e00 = wid * PE_TILE
        pltpu.async_copy(tab_hbm.at[rowidx.at[0]], bufA, semA)

        def pair(p, carry):
            ch0 = p * 2
            e0 = e00 + ch0 * CH
            pltpu.async_copy(tab_hbm.at[rowidx.at[ch0 + 1]], bufB, semB)
            pltpu.sync_copy(col_hbm.at[pl.ds(e0, CH)], idxcA)
            pltpu.sync_copy(col_hbm.at[pl.ds(e0 + CH, CH)], idxcB)
            pltpu.make_async_copy(tab_hbm.at[rowidx.at[ch0]], bufA,
                                  semA).wait()
            pltpu.sync_copy(bufA, acc_sh.at[idxcA], add=True)
            nxt = jnp.minimum(ch0 + 2, NCHT - 1)
            pltpu.async_copy(tab_hbm.at[rowidx.at[nxt]], bufA, semA)
            pltpu.make_async_copy(tab_hbm.at[rowidx.at[ch0 + 1]], bufB,
                                  semB).wait()
            pltpu.sync_copy(bufB, acc_sh.at[idxcB], add=True)
            return carry

        lax.fori_loop(0, NCHT // 2, pair, 0)
        pltpu.make_async_copy(tab_hbm.at[rowidx.at[NCHT - 1]], bufA,
                              semA).wait()
        plsc.subcore_barrier()
        pltpu.sync_copy(acc_sh.at[pl.ds(r0, ROWS16)],
                        out_hbm.at[cid, pl.ds(r0, ROWS16)])

    return k


def _make_sc_att(interpret=False):
    @functools.partial(
        pl.kernel,
        out_type=(
            jax.ShapeDtypeStruct((NC, EP), jnp.float32),
            jax.ShapeDtypeStruct((NC, NP2), jnp.float32),
        ),
        mesh=_mesh(),
        compiler_params=pltpu.CompilerParams(needs_layout_passes=False),
        scratch_types=[
            pltpu.VMEM((NP2,), jnp.float32),
            pltpu.VMEM((NP2,), jnp.float32),
            pltpu.VMEM((NP2,), jnp.float32),
            pltpu.VMEM((NP2,), jnp.float32),
            pltpu.VMEM((NCHC, CH), jnp.int32),
            pltpu.VMEM((NCHC, CH), jnp.int32),
            pltpu.VMEM((PE_CTILE,), jnp.float32),
            pltpu.VMEM((ROWS16,), jnp.float32),
            pltpu.VMEM((ROWS16,), jnp.float32),
            pltpu.VMEM((NP2,), jnp.float32),
            pltpu.VMEM_SHARED((NP2,), jnp.float32),
            pltpu.VMEM_SHARED((NP2,), jnp.float32),
            pltpu.SemaphoreType.DMA,
        ],
        interpret=interpret,
    )
    def k(att0_hbm, att1_hbm, att2_hbm, att3_hbm, row_hbm, col_hbm,
          alpha_hbm, dinv_hbm,
          a0_v, a1_v, b0_v, b1_v, rowidx, colidx, ew_all, degbuf, dinv1d,
          dinv_v, deg_sh, dinv_sh, semD):
        cid = lax.axis_index("c")
        sid = lax.axis_index("s")
        r0 = sid * ROWS16
        _fill1d(degbuf, ROWS16, 0.0)
        pltpu.sync_copy(degbuf, deg_sh.at[pl.ds(r0, ROWS16)])
        pltpu.sync_copy(att0_hbm, a0_v)
        pltpu.sync_copy(att1_hbm, a1_v)
        pltpu.sync_copy(att2_hbm, b0_v)
        pltpu.sync_copy(att3_hbm, b1_v)
        cb = sid * NCHC
        pltpu.sync_copy(row_hbm.at[pl.ds(cb, NCHC)], rowidx)
        pltpu.sync_copy(col_hbm.at[pl.ds(cb, NCHC)], colidx)
        plsc.subcore_barrier()

        base = sid * PE_CTILE
        i16 = _i16()
        is_c = (cid == 0)

        def ew_chunk(ch):
            for g in range(CH // 16):
                r16 = rowidx[ch, pl.ds(g * 16, 16)]
                c16 = colidx[ch, pl.ds(g * 16, 16)]
                a0 = plsc.load_gather(a0_v, [r16])
                a1 = plsc.load_gather(a1_v, [r16])
                b0 = plsc.load_gather(b0_v, [c16])
                b1 = plsc.load_gather(b1_v, [c16])
                l0 = a0 + b0
                l1 = a1 + b1
                m = jnp.maximum(l0, l1)
                x0 = jnp.exp(l0 - m)
                x1 = jnp.exp(l1 - m)
                s = x0 + x1
                ew = jnp.where(is_c, x0, x1) / s
                eid = base + ch * CH + g * 16 + i16
                ew = jnp.where(eid < E, ew,
                               jnp.where(eid < E + N, 1.0, 0.0))
                ew_all[pl.ds(ch * CH + g * 16, 16)] = ew

        def pass1(q, carry):
            for b in range(2):
                ew_chunk(q * 2 + b)
            for b in range(2):
                ch = q * 2 + b
                pltpu.async_copy(ew_all.at[pl.ds(ch * CH, CH)],
                                 deg_sh.at[rowidx.at[ch]], semD, add=True)
            for b in range(2):
                ch = q * 2 + b
                pltpu.make_async_copy(ew_all.at[pl.ds(ch * CH, CH)],
                                      deg_sh.at[rowidx.at[ch]], semD).wait()
            return carry

        lax.fori_loop(0, NCHC // 2, pass1, 0)
        plsc.subcore_barrier()

        pltpu.sync_copy(deg_sh.at[pl.ds(r0, ROWS16)], degbuf)
        for g in range(ROWS16 // 16):
            dg = degbuf[pl.ds(g * 16, 16)]
            y = jnp.where(dg > 0, _rsqrt16(dg), 0.0)
            dinv1d[pl.ds(g * 16, 16)] = y
        pltpu.sync_copy(dinv1d, dinv_sh.at[pl.ds(r0, ROWS16)])
        pltpu.sync_copy(dinv1d, dinv_hbm.at[cid, pl.ds(r0, ROWS16)])
        plsc.subcore_barrier()
        pltpu.sync_copy(dinv_sh, dinv_v)

        def pass2(ch, carry):
            for g in range(CH // 16):
                r16 = rowidx[ch, pl.ds(g * 16, 16)]
                dv = plsc.load_gather(dinv_v, [r16])
                o = ch * CH + g * 16
                ew_all[pl.ds(o, 16)] = ew_all[pl.ds(o, 16)] * dv
            return carry

        lax.fori_loop(0, NCHC, pass2, 0, unroll=2)
        pltpu.sync_copy(ew_all, alpha_hbm.at[cid, pl.ds(base, PE_CTILE)])

    return k


def _make_sc_conv_co(interpret=False):
    @functools.partial(
        pl.kernel,
        out_type=jax.ShapeDtypeStruct((NC, NP2, H), jnp.float32),
        mesh=_mesh(),
        compiler_params=pltpu.CompilerParams(needs_layout_passes=False),
        scratch_types=[
            pltpu.VMEM((CH,), jnp.int32),
            pltpu.VMEM((CH,), jnp.int32),
            pltpu.VMEM((CH,), jnp.float32),
            pltpu.VMEM((CH, H), jnp.float32),
            pltpu.VMEM_SHARED((NP2, H), jnp.float32),
            pltpu.SemaphoreType.DMA,
        ],
        interpret=interpret,
    )
    def k(xw_hbm, alpha_hbm, row_hbm, col_hbm,
          out_hbm, idxr, idxc, avmem, bufA, acc_sh, semA):
        cid = lax.axis_index("c")
        sid = lax.axis_index("s")
        _zero_rows(bufA, CH, H)
        r0 = sid * ROWS16
        for b in range(ROWS16 // CH):
            pltpu.sync_copy(bufA, acc_sh.at[pl.ds(r0 + b * CH, CH)])
        plsc.subcore_barrier()

        base = sid * PE_CTILE

        def body(ch, carry):
            e0 = base + ch * CH
            pltpu.sync_copy(row_hbm.at[pl.ds(e0, CH)], idxr)
            pltpu.sync_copy(col_hbm.at[pl.ds(e0, CH)], idxc)
            pltpu.sync_copy(alpha_hbm.at[cid, pl.ds(e0, CH)], avmem)
            pltpu.async_copy(xw_hbm.at[cid].at[idxr], bufA, semA).wait()

            def scale(r, carry2):
                al = plsc.load_gather(avmem, [jnp.full((16,), r, jnp.int32)])
                for j in range(H // 16):
                    bufA[r, pl.ds(j * 16, 16)] = bufA[r, pl.ds(j * 16, 16)] * al
                return carry2

            lax.fori_loop(0, CH, scale, 0, unroll=4)
            pltpu.sync_copy(bufA, acc_sh.at[idxc], add=True)
            return carry

        lax.fori_loop(0, NCHC, body, 0)
        plsc.subcore_barrier()
        pltpu.sync_copy(acc_sh.at[pl.ds(r0, ROWS16)],
                        out_hbm.at[cid, pl.ds(r0, ROWS16)])

    return k


def _bn(x):
    m = jnp.mean(x, axis=0)
    xc = x - m
    v = jnp.mean(xc * xc, axis=0)
    return xc * lax.rsqrt(v + EPS) + 1e-4


def _dinv_from(deg_ref):
    d = deg_ref[...]
    ds_ = (d[0] + d[1]).reshape(NP2, 1)
    return jnp.where(ds_ > 0, lax.rsqrt(ds_), 0.0)[:N]


def _padz(x):
    return jnp.concatenate([x, jnp.zeros((NP2 - N, x.shape[1]), x.dtype)], 0)


def _make_tc_layer1(interpret=False):
    def body(x_ref, wf_ref, w0_ref, deg_ref, o_ref):
        xb = _bn(x_ref[...])
        h0 = jnp.maximum(jnp.dot(xb, wf_ref[...],
                                 preferred_element_type=jnp.float32), 0.0)
        xw = jnp.dot(_bn(h0), w0_ref[...], preferred_element_type=jnp.float32)
        o_ref[...] = _padz(_dinv_from(deg_ref) * xw)

    return pl.pallas_call(
        body, out_shape=jax.ShapeDtypeStruct((NP2, H), jnp.float32),
        interpret=interpret)


def _make_tc_layer(interpret=False):
    def body(t_ref, deg_ref, b_ref, w_ref, o_ref):
        dinv = _dinv_from(deg_ref)
        s = t_ref[0, :N, :] + t_ref[1, :N, :]
        h = jnp.maximum(dinv * s + b_ref[...], 0.0)
        xw = jnp.dot(_bn(h), w_ref[...], preferred_element_type=jnp.float32)
        o_ref[...] = _padz(dinv * xw)

    return pl.pallas_call(
        body, out_shape=jax.ShapeDtypeStruct((NP2, H), jnp.float32),
        interpret=interpret)


def _make_tc_att(interpret=False):
    def body(t_ref, deg_ref, b_ref, weaT_ref, bea_ref, wna_ref, bna_ref,
             wctx_ref, wobj_ref, a0_ref, a1_ref, b0_ref, b1_ref, xw_ref):
        dinv = _dinv_from(deg_ref)
        s = t_ref[0, :N, :] + t_ref[1, :N, :]
        h3 = jnp.maximum(dinv * s + b_ref[...], 0.0)
        weaT = weaT_ref[...]
        bea = bea_ref[...]
        zrow = jnp.zeros((NP2 - N,), jnp.float32)

        def rowdot(w_1x128):
            r = lax.dot_general(w_1x128, h3, (((1,), (1,)), ((), ())),
                                preferred_element_type=jnp.float32)
            return r.reshape(N)

        a0_ref[...] = jnp.concatenate([rowdot(weaT[0:1, :H]) + bea[0], zrow])
        a1_ref[...] = jnp.concatenate([rowdot(weaT[1:2, :H]) + bea[1], zrow])
        b0_ref[...] = jnp.concatenate([rowdot(weaT[0:1, H:]), zrow])
        b1_ref[...] = jnp.concatenate([rowdot(weaT[1:2, H:]), zrow])
        nl = jnp.dot(h3, wna_ref[...], preferred_element_type=jnp.float32) \
            + bna_ref[...]
        m = jnp.max(nl, axis=1, keepdims=True)
        ex = jnp.exp(nl - m)
        nat = ex / jnp.sum(ex, axis=1, keepdims=True)
        xw_ref[0] = _padz(jnp.dot(_bn(nat[:, 0:1] * h3), wctx_ref[...],
                                  preferred_element_type=jnp.float32))
        xw_ref[1] = _padz(jnp.dot(_bn(nat[:, 1:2] * h3), wobj_ref[...],
                                  preferred_element_type=jnp.float32))

    return pl.pallas_call(
        body,
        out_shape=(jax.ShapeDtypeStruct((NP2,), jnp.float32),
                   jax.ShapeDtypeStruct((NP2,), jnp.float32),
                   jax.ShapeDtypeStruct((NP2,), jnp.float32),
                   jax.ShapeDtypeStruct((NP2,), jnp.float32),
                   jax.ShapeDtypeStruct((NC, NP2, H), jnp.float32)),
        interpret=interpret)


def _make_tc_final(interpret=False):
    def body(t_ref, dinv_ref, bias_ref, batch_ref, w1c, b1c, w2c, b2c,
             w1o, b1o, w2o, b2o, w1co, b1co, w2co, b2co,
             oc_ref, oo_ref, oco_ref):
        bt = batch_ref[...]
        onehotT = (lax.broadcasted_iota(jnp.int32, (G, N), 0)
                   == jnp.broadcast_to(bt, (G, N))).astype(jnp.float32)
        dinv = dinv_ref[...]
        bias = bias_ref[...]

        def chan_pool(c2):
            t = t_ref[c2, :N, :]
            dv = dinv[c2].reshape(NP2, 1)[:N]
            xr = jnp.maximum(dv * t + bias[c2], 0.0)
            return jnp.dot(onehotT, xr, preferred_element_type=jnp.float32)

        pc = chan_pool(0)
        po = chan_pool(1)

        def readout(z, w1, b1, w2, b2):
            z = _bn(z)
            z = jnp.maximum(jnp.dot(z, w1[...],
                                    preferred_element_type=jnp.float32)
                            + b1[...], 0.0)
            z = _bn(z)
            z = jnp.dot(z, w2[...], preferred_element_type=jnp.float32) \
                + b2[...]
            m = jnp.max(z, axis=1, keepdims=True)
            return z - (m + jnp.log(jnp.sum(jnp.exp(z - m), axis=1,
                                            keepdims=True)))

        oc_ref[...] = readout(pc, w1c, b1c, w2c, b2c)
        oo_ref[...] = readout(po, w1o, b1o, w2o, b2o)
        oco_ref[...] = readout(pc + po, w1co, b1co, w2co, b2co)

    return pl.pallas_call(
        body,
        out_shape=(jax.ShapeDtypeStruct((G, C), jnp.float32),
                   jax.ShapeDtypeStruct((G, C), jnp.float32),
                   jax.ShapeDtypeStruct((G, C), jnp.float32)),
        interpret=interpret)


_sc_cache = {}


def _sc(name, maker):
    if name not in _sc_cache:
        _sc_cache[name] = maker()
    return _sc_cache[name]


_tc_layer1 = _make_tc_layer1()
_tc_layer = _make_tc_layer()
_tc_att = _make_tc_att()
_tc_final = _make_tc_final()


def _kernel_staged(x, edge_index, batch, W_feat, W_convs, b_convs, W_ea, b_ea,
                   W_na, b_na, W_ctx, b_ctx, W_obj, b_obj, fc1_c_W, fc1_c_b,
                   fc2_c_W, fc2_c_b, fc1_o_W, fc1_o_b, fc2_o_W, fc2_o_b,
                   fc1_co_W, fc1_co_b, fc2_co_W, fc2_co_b, depth=99):
    loops = jnp.arange(N, dtype=jnp.int32)
    pade = EP - (E + N)
    padi = PADSTART + (jnp.arange(pade, dtype=jnp.int32) % 64)
    row_full = jnp.concatenate([edge_index[0], loops, padi])
    col_full = jnp.concatenate([edge_index[1], loops, padi])
    row2d = row_full.reshape(EP // CH, CH)
    col2d = col_full.reshape(EP // CH, CH)
    bias_st = jnp.stack([b_ctx, b_obj])
    sc_prep = _sc("prep", _make_sc_prep)
    sc_conv = _sc("conv", _make_sc_conv)
    sc_att = _sc("att", _make_sc_att)
    sc_conv_co = _sc("conv_co", _make_sc_conv_co)
    deg = sc_prep(row2d)
    if depth <= 1:
        return deg
    t = _tc_layer1(x, W_feat, W_convs[0], deg)
    T = sc_conv(t, row2d, col_full)
    if depth <= 2:
        return T
    t = _tc_layer(T, deg, b_convs[0], W_convs[1])
    T = sc_conv(t, row2d, col_full)
    t = _tc_layer(T, deg, b_convs[1], W_convs[2])
    T = sc_conv(t, row2d, col_full)
    if depth <= 3:
        return T
    a0, a1, b0, b1, xw_st = _tc_att(T, deg, b_convs[2], W_ea.T, b_ea,
                                    W_na, b_na, W_ctx, W_obj)
    alpha, dinv_co = sc_att(a0, a1, b0, b1, row2d, col2d)
    if depth <= 4:
        return alpha, dinv_co
    Tco = sc_conv_co(xw_st, alpha, row_full, col_full)
    if depth <= 5:
        return Tco
    return _tc_final(Tco, dinv_co, bias_st, batch,
                     fc1_c_W, fc1_c_b, fc2_c_W, fc2_c_b,
                     fc1_o_W, fc1_o_b, fc2_o_W, fc2_o_b,
                     fc1_co_W, fc1_co_b, fc2_co_W, fc2_co_b)


def kernel(x, edge_index, batch, W_feat, W_convs, b_convs, W_ea, b_ea, W_na,
           b_na, W_ctx, b_ctx, W_obj, b_obj, fc1_c_W, fc1_c_b, fc2_c_W,
           fc2_c_b, fc1_o_W, fc1_o_b, fc2_o_W, fc2_o_b, fc1_co_W, fc1_co_b,
           fc2_co_W, fc2_co_b):
    loops = jnp.arange(N, dtype=jnp.int32)
    pade = EP - (E + N)
    padi = PADSTART + (jnp.arange(pade, dtype=jnp.int32) % 64)
    row_full = jnp.concatenate([edge_index[0], loops, padi])
    col_full = jnp.concatenate([edge_index[1], loops, padi])
    row2d = row_full.reshape(EP // CH, CH)
    col2d = col_full.reshape(EP // CH, CH)
    bias_st = jnp.stack([b_ctx, b_obj])

    sc_prep = _sc("prep", _make_sc_prep)
    sc_conv = _sc("conv", _make_sc_conv)
    sc_att = _sc("att", _make_sc_att)
    sc_conv_co = _sc("conv_co", _make_sc_conv_co)

    deg = sc_prep(row2d)
    t = _tc_layer1(x, W_feat, W_convs[0], deg)
    T = sc_conv(t, row2d, col_full)
    t = _tc_layer(T, deg, b_convs[0], W_convs[1])
    T = sc_conv(t, row2d, col_full)
    t = _tc_layer(T, deg, b_convs[1], W_convs[2])
    T = sc_conv(t, row2d, col_full)
    a0, a1, b0, b1, xw_st = _tc_att(T, deg, b_convs[2], W_ea.T, b_ea,
                                    W_na, b_na, W_ctx, W_obj)
    alpha, dinv_co = sc_att(a0, a1, b0, b1, row2d, col2d)
    Tco = sc_conv_co(xw_st, alpha, row_full, col_full)
    return _tc_final(Tco, dinv_co, bias_st, batch,
                     fc1_c_W, fc1_c_b, fc2_c_W, fc2_c_b,
                     fc1_o_W, fc1_o_b, fc2_o_W, fc2_o_b,
                     fc1_co_W, fc1_co_b, fc2_co_W, fc2_co_b)
